# baseline (speedup 1.0000x reference)
import functools

import jax
import jax.numpy as jnp
from jax.experimental import pallas as pl
from jax.experimental.pallas import tpu as pltpu

POS_DIM = 3
EPS = 1e-5
SIGMA = 0.5


def _compw_body(ew_ref, kw_ref, out_ref):
    out_ref[0] = jnp.dot(ew_ref[...], kw_ref[0],
                         preferred_element_type=jnp.float32)


def _make_comp_w(weights, kw):
    n, c = weights.shape
    k = kw.shape[0]
    return pl.pallas_call(
        _compw_body,
        grid=(k,),
        in_specs=[
            pl.BlockSpec((n, c), lambda i: (0, 0)),
            pl.BlockSpec((1, c, c), lambda i: (i, 0, 0)),
        ],
        out_specs=pl.BlockSpec((1, n, c), lambda i: (i, 0, 0)),
        out_shape=jax.ShapeDtypeStruct((k, n, c), jnp.float32),
    )(weights, kw)


def _sample_body(active_ref, jeff_ref, qpos_ref, qb_ref, cposT_ref, cb_ref,
                 cw_ref, kpos_ref, kposT_ref, out_ref, *, inv2s2, k):
    i = pl.program_id(0)
    j = pl.program_id(1)

    @pl.when(j == 0)
    def _():
        out_ref[...] = jnp.zeros_like(out_ref)

    @pl.when(active_ref[i, j] != 0)
    def _():
        zp = qpos_ref[...]
        epT = cposT_ref[...]
        kp = kpos_ref[:, 0, :]
        kpT = kposT_ref[...]
        zb = qb_ref[0]
        eb = cb_ref[0]
        mask = zb.T == eb
        base = jnp.zeros_like(mask, dtype=jnp.float32)
        zt = jnp.zeros((zp.shape[0], k), jnp.float32)
        et = jnp.zeros((k, epT.shape[1]), jnp.float32)
        for d in range(POS_DIM):
            diff = zp[:, d:d + 1] - epT[d:d + 1, :]
            base = base + diff * diff
            zt = zt + zp[:, d:d + 1] * kpT[d:d + 1, :]
            et = et + kp[:, d:d + 1] * epT[d:d + 1, :]
        basem = jnp.where(mask, base, 1.0e30)
        t2 = jnp.sum(kp * kp, axis=1)[:, None]
        kern = jnp.concatenate(
            [jnp.exp(-inv2s2 * ((basem - 2.0 * zt[:, kk:kk + 1])
                                + (2.0 * et[kk:kk + 1, :] + t2[kk, 0])))
             for kk in range(k)],
            axis=1)
        cw = cw_ref[...].reshape(k * cw_ref.shape[1], cw_ref.shape[2])
        out_ref[...] += jnp.dot(kern, cw, preferred_element_type=jnp.float32)


def _block_meta(q_batch, c_batch, bz, be):
    gi = q_batch.shape[0] // bz
    gj = c_batch.shape[0] // be
    qb = q_batch.reshape(gi, bz)
    cb = c_batch.reshape(gj, be)
    qmin, qmax = qb[:, 0], qb[:, -1]
    cmin, cmax = cb[:, 0], cb[:, -1]
    active = ((cmin[None, :] <= qmax[:, None])
              & (qmin[:, None] <= cmax[None, :])).astype(jnp.int32)
    idx = jnp.where(active == 1, jnp.arange(gj, dtype=jnp.int32)[None, :], -1)
    jeff = jnp.maximum(jax.lax.cummax(idx, axis=1), 0).astype(jnp.int32)
    return active, jeff


def _sample(q_pos, q_batch, c_pos, c_batch, comp_w, kpos, sigma,
            bz=256, be=256):
    nq = q_pos.shape[0]
    nc = c_pos.shape[0]
    k, _, c = comp_w.shape
    gi, gj = nq // bz, nc // be
    active, jeff = _block_meta(q_batch, c_batch, bz, be)
    qb = q_batch.reshape(gi, 1, bz)
    cb = c_batch.reshape(gj, 1, be)
    c_posT = c_pos.T
    kpos3 = kpos.reshape(k, 1, POS_DIM)
    kposT = kpos.T
    grid_spec = pltpu.PrefetchScalarGridSpec(
        num_scalar_prefetch=2,
        grid=(gi, gj),
        in_specs=[
            pl.BlockSpec((bz, POS_DIM), lambda i, j, act, jef: (i, 0)),
            pl.BlockSpec((1, 1, bz), lambda i, j, act, jef: (i, 0, 0)),
            pl.BlockSpec((POS_DIM, be),
                         lambda i, j, act, jef: (0, jef[i, j])),
            pl.BlockSpec((1, 1, be),
                         lambda i, j, act, jef: (jef[i, j], 0, 0)),
            pl.BlockSpec((k, be, c),
                         lambda i, j, act, jef: (0, jef[i, j], 0)),
            pl.BlockSpec((k, 1, POS_DIM), lambda i, j, act, jef: (0, 0, 0)),
            pl.BlockSpec((POS_DIM, k), lambda i, j, act, jef: (0, 0)),
        ],
        out_specs=pl.BlockSpec((bz, c), lambda i, j, act, jef: (i, 0)),
    )
    return pl.pallas_call(
        functools.partial(_sample_body, inv2s2=1.0 / (2.0 * sigma * sigma),
                          k=k),
        grid_spec=grid_spec,
        out_shape=jax.ShapeDtypeStruct((nq, c), jnp.float32),
    )(active, jeff, q_pos, qb, c_posT, cb, comp_w, kpos3, kposT)


def _bnadd_body(x_ref, g_ref, b_ref, base_ref, out_ref):
    x = x_ref[...]
    x = jnp.where(x >= 0, x, 0.01 * x)
    m = jnp.mean(x, axis=0, keepdims=True)
    v = jnp.mean((x - m) ** 2, axis=0, keepdims=True)
    out_ref[...] = (base_ref[...]
                    + (x - m) * jax.lax.rsqrt(v + EPS) * g_ref[...]
                    + b_ref[...])


def _bnadd(x, gamma, beta, base):
    c = x.shape[-1]
    return pl.pallas_call(
        _bnadd_body,
        out_shape=jax.ShapeDtypeStruct(x.shape, jnp.float32),
    )(x, gamma.reshape(1, c), beta.reshape(1, c), base)


def _mlp_body(zw_ref, zpos_ref, w1_ref, b1_ref, g_ref, bt_ref,
              w2p_ref, w2w_ref, b2p_ref, b2w_ref, opos_ref, ow_ref):
    zw = zw_ref[...]
    h = jnp.dot(zw, w1_ref[...], preferred_element_type=jnp.float32)
    h = h + b1_ref[...]
    h = jnp.where(h >= 0, h, 0.01 * h)
    m = jnp.mean(h, axis=0, keepdims=True)
    v = jnp.mean((h - m) ** 2, axis=0, keepdims=True)
    h = (h - m) * jax.lax.rsqrt(v + EPS) * g_ref[...] + bt_ref[...]
    dpos = jnp.dot(h, w2p_ref[...], preferred_element_type=jnp.float32)
    dpos = dpos + b2p_ref[...]
    dw = jnp.dot(h, w2w_ref[...], preferred_element_type=jnp.float32)
    dw = dw + b2w_ref[...]
    opos_ref[...] = zpos_ref[...] + dpos[:, :POS_DIM]
    ow_ref[...] = zw + dw


def kernel(z_positions, z_weights, z_batch, e_positions, e_weights, e_batch,
           cross_kpos, cross_kw, norm_cross_gamma, norm_cross_beta,
           self_kpos, self_kw, norm_self_gamma, norm_self_beta,
           mlp_w1, mlp_b1, mlp_bn_gamma, mlp_bn_beta, mlp_w2, mlp_b2):
    nz, c = z_weights.shape
    c_mlp = mlp_w1.shape[1]

    cw1 = _make_comp_w(e_weights, cross_kw)
    s1 = _sample(z_positions, z_batch, e_positions, e_batch, cw1,
                 cross_kpos, SIGMA)
    zw = _bnadd(s1, norm_cross_gamma, norm_cross_beta, z_weights)

    cw2 = _make_comp_w(zw, self_kw)
    s2 = _sample(z_positions, z_batch, z_positions, z_batch, cw2,
                 self_kpos, SIGMA)
    zw2 = _bnadd(s2, norm_self_gamma, norm_self_beta, zw)

    w2_pos = jnp.zeros((c_mlp, c), jnp.float32).at[:, :POS_DIM].set(
        mlp_w2[:, :POS_DIM])
    b2_pos = jnp.zeros((1, c), jnp.float32).at[0, :POS_DIM].set(
        mlp_b2[:POS_DIM])
    w2_w = mlp_w2[:, POS_DIM:]
    b2_w = mlp_b2[POS_DIM:].reshape(1, c)

    out_pos, out_w = pl.pallas_call(
        _mlp_body,
        out_shape=(
            jax.ShapeDtypeStruct((nz, POS_DIM), jnp.float32),
            jax.ShapeDtypeStruct((nz, c), jnp.float32),
        ),
    )(zw2, z_positions, mlp_w1, mlp_b1.reshape(1, c_mlp),
      mlp_bn_gamma.reshape(1, c_mlp), mlp_bn_beta.reshape(1, c_mlp),
      w2_pos, w2_w, b2_pos, b2_w)
    return out_pos, out_w


# speedup vs baseline: 1.0035x; 1.0035x over previous
import functools

import jax
import jax.numpy as jnp
from jax.experimental import pallas as pl
from jax.experimental.pallas import tpu as pltpu

POS_DIM = 3
EPS = 1e-5
SIGMA = 0.5


def _compw_body(ew_ref, kw_ref, out_ref):
    out_ref[0] = jnp.dot(ew_ref[...], kw_ref[0],
                         preferred_element_type=jnp.float32)


def _make_comp_w(weights, kw):
    n, c = weights.shape
    k = kw.shape[0]
    return pl.pallas_call(
        _compw_body,
        grid=(k,),
        in_specs=[
            pl.BlockSpec((n, c), lambda i: (0, 0)),
            pl.BlockSpec((1, c, c), lambda i: (i, 0, 0)),
        ],
        out_specs=pl.BlockSpec((1, n, c), lambda i: (i, 0, 0)),
        out_shape=jax.ShapeDtypeStruct((k, n, c), jnp.float32),
    )(weights, kw)


def _sample_body(active_ref, jeff_ref, qpos_ref, qb_ref, cposT_ref, cb_ref,
                 cw_ref, kpos_ref, out_ref, *, inv2s2, k):
    i = pl.program_id(0)
    j = pl.program_id(1)

    @pl.when(j == 0)
    def _():
        out_ref[...] = jnp.zeros_like(out_ref)

    @pl.when(active_ref[i, j] != 0)
    def _():
        zp = qpos_ref[...]
        epT = cposT_ref[...]
        kp = kpos_ref[:, 0, :]
        zb = qb_ref[0]
        eb = cb_ref[0]
        mask = zb.T == eb
        z2 = jnp.sum(zp * zp, axis=1)[:, None]
        slabs = []
        for kk in range(k):
            ptT = epT + kp[kk, :][:, None]
            d2 = (z2 + jnp.sum(ptT * ptT, axis=0)[None, :]
                  - 2.0 * jnp.dot(zp, ptT))
            slabs.append(jnp.where(mask, jnp.exp(-inv2s2 * d2), 0.0))
        kern = jnp.concatenate(slabs, axis=1)
        cw = cw_ref[...].reshape(k * cw_ref.shape[1], cw_ref.shape[2])
        out_ref[...] += jnp.dot(kern, cw, preferred_element_type=jnp.float32)


def _block_meta(q_batch, c_batch, bz, be):
    gi = q_batch.shape[0] // bz
    gj = c_batch.shape[0] // be
    qb = q_batch.reshape(gi, bz)
    cb = c_batch.reshape(gj, be)
    qmin, qmax = qb[:, 0], qb[:, -1]
    cmin, cmax = cb[:, 0], cb[:, -1]
    active = ((cmin[None, :] <= qmax[:, None])
              & (qmin[:, None] <= cmax[None, :])).astype(jnp.int32)
    idx = jnp.where(active == 1, jnp.arange(gj, dtype=jnp.int32)[None, :], -1)
    jeff = jnp.maximum(jax.lax.cummax(idx, axis=1), 0).astype(jnp.int32)
    return active, jeff


def _sample(q_pos, q_batch, c_pos, c_batch, comp_w, kpos, sigma,
            bz=256, be=256):
    nq = q_pos.shape[0]
    nc = c_pos.shape[0]
    k, _, c = comp_w.shape
    gi, gj = nq // bz, nc // be
    active, jeff = _block_meta(q_batch, c_batch, bz, be)
    qb = q_batch.reshape(gi, 1, bz)
    cb = c_batch.reshape(gj, 1, be)
    c_posT = c_pos.T
    kpos3 = kpos.reshape(k, 1, POS_DIM)
    grid_spec = pltpu.PrefetchScalarGridSpec(
        num_scalar_prefetch=2,
        grid=(gi, gj),
        in_specs=[
            pl.BlockSpec((bz, POS_DIM), lambda i, j, act, jef: (i, 0)),
            pl.BlockSpec((1, 1, bz), lambda i, j, act, jef: (i, 0, 0)),
            pl.BlockSpec((POS_DIM, be),
                         lambda i, j, act, jef: (0, jef[i, j])),
            pl.BlockSpec((1, 1, be),
                         lambda i, j, act, jef: (jef[i, j], 0, 0)),
            pl.BlockSpec((k, be, c),
                         lambda i, j, act, jef: (0, jef[i, j], 0)),
            pl.BlockSpec((k, 1, POS_DIM), lambda i, j, act, jef: (0, 0, 0)),
        ],
        out_specs=pl.BlockSpec((bz, c), lambda i, j, act, jef: (i, 0)),
    )
    return pl.pallas_call(
        functools.partial(_sample_body, inv2s2=1.0 / (2.0 * sigma * sigma),
                          k=k),
        grid_spec=grid_spec,
        out_shape=jax.ShapeDtypeStruct((nq, c), jnp.float32),
    )(active, jeff, q_pos, qb, c_posT, cb, comp_w, kpos3)


def _bnadd_body(x_ref, g_ref, b_ref, base_ref, out_ref):
    x = x_ref[...]
    x = jnp.where(x >= 0, x, 0.01 * x)
    m = jnp.mean(x, axis=0, keepdims=True)
    v = jnp.mean((x - m) ** 2, axis=0, keepdims=True)
    out_ref[...] = (base_ref[...]
                    + (x - m) * jax.lax.rsqrt(v + EPS) * g_ref[...]
                    + b_ref[...])


def _bnadd(x, gamma, beta, base):
    c = x.shape[-1]
    return pl.pallas_call(
        _bnadd_body,
        out_shape=jax.ShapeDtypeStruct(x.shape, jnp.float32),
    )(x, gamma.reshape(1, c), beta.reshape(1, c), base)


def _mlp_body(zw_ref, zpos_ref, w1_ref, b1_ref, g_ref, bt_ref,
              w2p_ref, w2w_ref, b2p_ref, b2w_ref, opos_ref, ow_ref):
    zw = zw_ref[...]
    h = jnp.dot(zw, w1_ref[...], preferred_element_type=jnp.float32)
    h = h + b1_ref[...]
    h = jnp.where(h >= 0, h, 0.01 * h)
    m = jnp.mean(h, axis=0, keepdims=True)
    v = jnp.mean((h - m) ** 2, axis=0, keepdims=True)
    h = (h - m) * jax.lax.rsqrt(v + EPS) * g_ref[...] + bt_ref[...]
    dpos = jnp.dot(h, w2p_ref[...], preferred_element_type=jnp.float32)
    dpos = dpos + b2p_ref[...]
    dw = jnp.dot(h, w2w_ref[...], preferred_element_type=jnp.float32)
    dw = dw + b2w_ref[...]
    opos_ref[...] = zpos_ref[...] + dpos[:, :POS_DIM]
    ow_ref[...] = zw + dw


def kernel(z_positions, z_weights, z_batch, e_positions, e_weights, e_batch,
           cross_kpos, cross_kw, norm_cross_gamma, norm_cross_beta,
           self_kpos, self_kw, norm_self_gamma, norm_self_beta,
           mlp_w1, mlp_b1, mlp_bn_gamma, mlp_bn_beta, mlp_w2, mlp_b2):
    nz, c = z_weights.shape
    c_mlp = mlp_w1.shape[1]

    cw1 = _make_comp_w(e_weights, cross_kw)
    s1 = _sample(z_positions, z_batch, e_positions, e_batch, cw1,
                 cross_kpos, SIGMA)
    zw = _bnadd(s1, norm_cross_gamma, norm_cross_beta, z_weights)

    cw2 = _make_comp_w(zw, self_kw)
    s2 = _sample(z_positions, z_batch, z_positions, z_batch, cw2,
                 self_kpos, SIGMA)
    zw2 = _bnadd(s2, norm_self_gamma, norm_self_beta, zw)

    w2_pos = jnp.zeros((c_mlp, c), jnp.float32).at[:, :POS_DIM].set(
        mlp_w2[:, :POS_DIM])
    b2_pos = jnp.zeros((1, c), jnp.float32).at[0, :POS_DIM].set(
        mlp_b2[:POS_DIM])
    w2_w = mlp_w2[:, POS_DIM:]
    b2_w = mlp_b2[POS_DIM:].reshape(1, c)

    out_pos, out_w = pl.pallas_call(
        _mlp_body,
        out_shape=(
            jax.ShapeDtypeStruct((nz, POS_DIM), jnp.float32),
            jax.ShapeDtypeStruct((nz, c), jnp.float32),
        ),
    )(zw2, z_positions, mlp_w1, mlp_b1.reshape(1, c_mlp),
      mlp_bn_gamma.reshape(1, c_mlp), mlp_bn_beta.reshape(1, c_mlp),
      w2_pos, w2_w, b2_pos, b2_w)
    return out_pos, out_w


# speedup vs baseline: 1.0563x; 1.0526x over previous
import functools

import jax
import jax.numpy as jnp
from jax.experimental import pallas as pl
from jax.experimental.pallas import tpu as pltpu

POS_DIM = 3
EPS = 1e-5
SIGMA = 0.5


def _compw_body(ew_ref, kw_ref, out_ref):
    out_ref[0] = jnp.dot(ew_ref[...], kw_ref[0],
                         preferred_element_type=jnp.float32)


def _make_comp_w(weights, kw):
    n, c = weights.shape
    k = kw.shape[0]
    return pl.pallas_call(
        _compw_body,
        grid=(k,),
        in_specs=[
            pl.BlockSpec((n, c), lambda i: (0, 0)),
            pl.BlockSpec((1, c, c), lambda i: (i, 0, 0)),
        ],
        out_specs=pl.BlockSpec((1, n, c), lambda i: (i, 0, 0)),
        out_shape=jax.ShapeDtypeStruct((k, n, c), jnp.float32),
    )(weights, kw)


def _sample_body(active_ref, jeff_ref, qpos_ref, qb_ref, cposT_ref, cb_ref,
                 cw_ref, kpos_ref, out_ref, *, inv2s2, k):
    i = pl.program_id(0)
    j = pl.program_id(1)

    @pl.when(j == 0)
    def _():
        out_ref[...] = jnp.zeros_like(out_ref)

    @pl.when(active_ref[i, j] != 0)
    def _():
        zp = qpos_ref[...]
        epT = cposT_ref[...]
        kp = kpos_ref[:, 0, :]
        zb = qb_ref[0]
        eb = cb_ref[0]
        mask = zb.T == eb
        z2 = jnp.sum(zp * zp, axis=1)[:, None]
        slabs = []
        for kk in range(k):
            ptT = epT + kp[kk, :][:, None]
            d2 = (z2 + jnp.sum(ptT * ptT, axis=0)[None, :]
                  - 2.0 * jnp.dot(zp, ptT))
            slabs.append(jnp.where(mask, jnp.exp(-inv2s2 * d2), 0.0))
        kern = jnp.concatenate(slabs, axis=1)
        cw = cw_ref[...].reshape(k * cw_ref.shape[1], cw_ref.shape[2])
        out_ref[...] += jnp.dot(kern, cw, preferred_element_type=jnp.float32)


def _block_meta(q_batch, c_batch, bz, be):
    gi = q_batch.shape[0] // bz
    gj = c_batch.shape[0] // be
    qb = q_batch.reshape(gi, bz)
    cb = c_batch.reshape(gj, be)
    qmin, qmax = qb[:, 0], qb[:, -1]
    cmin, cmax = cb[:, 0], cb[:, -1]
    active = ((cmin[None, :] <= qmax[:, None])
              & (qmin[:, None] <= cmax[None, :])).astype(jnp.int32)
    idx = jnp.where(active == 1, jnp.arange(gj, dtype=jnp.int32)[None, :], -1)
    jeff = jnp.maximum(jax.lax.cummax(idx, axis=1), 0).astype(jnp.int32)
    return active, jeff


def _sample(q_pos, q_batch, c_pos, c_batch, comp_w, kpos, sigma,
            bz=256, be=256):
    nq = q_pos.shape[0]
    nc = c_pos.shape[0]
    k, _, c = comp_w.shape
    gi, gj = nq // bz, nc // be
    active, jeff = _block_meta(q_batch, c_batch, bz, be)
    qb = q_batch.reshape(gi, 1, bz)
    cb = c_batch.reshape(gj, 1, be)
    c_posT = c_pos.T
    kpos3 = kpos.reshape(k, 1, POS_DIM)
    grid_spec = pltpu.PrefetchScalarGridSpec(
        num_scalar_prefetch=2,
        grid=(gi, gj),
        in_specs=[
            pl.BlockSpec((bz, POS_DIM), lambda i, j, act, jef: (i, 0)),
            pl.BlockSpec((1, 1, bz), lambda i, j, act, jef: (i, 0, 0)),
            pl.BlockSpec((POS_DIM, be),
                         lambda i, j, act, jef: (0, jef[i, j])),
            pl.BlockSpec((1, 1, be),
                         lambda i, j, act, jef: (jef[i, j], 0, 0)),
            pl.BlockSpec((k, be, c),
                         lambda i, j, act, jef: (0, jef[i, j], 0)),
            pl.BlockSpec((k, 1, POS_DIM), lambda i, j, act, jef: (0, 0, 0)),
        ],
        out_specs=pl.BlockSpec((bz, c), lambda i, j, act, jef: (i, 0)),
    )
    return pl.pallas_call(
        functools.partial(_sample_body, inv2s2=1.0 / (2.0 * sigma * sigma),
                          k=k),
        grid_spec=grid_spec,
        out_shape=jax.ShapeDtypeStruct((nq, c), jnp.float32),
    )(active, jeff, q_pos, qb, c_posT, cb, comp_w, kpos3)


def _bn(x, gamma, beta):
    x = jnp.where(x >= 0, x, 0.01 * x)
    m = jnp.mean(x, axis=0, keepdims=True)
    v = jnp.mean((x - m) ** 2, axis=0, keepdims=True)
    return (x - m) * jax.lax.rsqrt(v + EPS) * gamma + beta


def _bn_compw_body(x_ref, g_ref, b_ref, base_ref, kw_ref, zw_ref, cw_ref,
                   *, k):
    zw = base_ref[...] + _bn(x_ref[...], g_ref[...], b_ref[...])
    zw_ref[...] = zw
    for kk in range(k):
        cw_ref[kk] = jnp.dot(zw, kw_ref[kk],
                             preferred_element_type=jnp.float32)


def _bn_compw(x, gamma, beta, base, kw):
    n, c = x.shape
    k = kw.shape[0]
    return pl.pallas_call(
        functools.partial(_bn_compw_body, k=k),
        out_shape=(
            jax.ShapeDtypeStruct((n, c), jnp.float32),
            jax.ShapeDtypeStruct((k, n, c), jnp.float32),
        ),
    )(x, gamma.reshape(1, c), beta.reshape(1, c), base, kw)


def _mlp_body(x_ref, g2_ref, b2s_ref, zwprev_ref, zpos_ref, w1_ref, b1_ref,
              g_ref, bt_ref, w2p_ref, w2w_ref, b2p_ref, b2w_ref,
              opos_ref, ow_ref):
    zw = zwprev_ref[...] + _bn(x_ref[...], g2_ref[...], b2s_ref[...])
    h = jnp.dot(zw, w1_ref[...], preferred_element_type=jnp.float32)
    h = h + b1_ref[...]
    h = jnp.where(h >= 0, h, 0.01 * h)
    m = jnp.mean(h, axis=0, keepdims=True)
    v = jnp.mean((h - m) ** 2, axis=0, keepdims=True)
    h = (h - m) * jax.lax.rsqrt(v + EPS) * g_ref[...] + bt_ref[...]
    dpos = jnp.dot(h, w2p_ref[...], preferred_element_type=jnp.float32)
    dpos = dpos + b2p_ref[...]
    dw = jnp.dot(h, w2w_ref[...], preferred_element_type=jnp.float32)
    dw = dw + b2w_ref[...]
    opos_ref[...] = zpos_ref[...] + dpos[:, :POS_DIM]
    ow_ref[...] = zw + dw


def kernel(z_positions, z_weights, z_batch, e_positions, e_weights, e_batch,
           cross_kpos, cross_kw, norm_cross_gamma, norm_cross_beta,
           self_kpos, self_kw, norm_self_gamma, norm_self_beta,
           mlp_w1, mlp_b1, mlp_bn_gamma, mlp_bn_beta, mlp_w2, mlp_b2):
    nz, c = z_weights.shape
    c_mlp = mlp_w1.shape[1]

    cw1 = _make_comp_w(e_weights, cross_kw)
    s1 = _sample(z_positions, z_batch, e_positions, e_batch, cw1,
                 cross_kpos, SIGMA)
    zw, cw2 = _bn_compw(s1, norm_cross_gamma, norm_cross_beta, z_weights,
                        self_kw)
    s2 = _sample(z_positions, z_batch, z_positions, z_batch, cw2,
                 self_kpos, SIGMA)

    w2_pos = jnp.zeros((c_mlp, c), jnp.float32).at[:, :POS_DIM].set(
        mlp_w2[:, :POS_DIM])
    b2_pos = jnp.zeros((1, c), jnp.float32).at[0, :POS_DIM].set(
        mlp_b2[:POS_DIM])
    w2_w = mlp_w2[:, POS_DIM:]
    b2_w = mlp_b2[POS_DIM:].reshape(1, c)

    out_pos, out_w = pl.pallas_call(
        _mlp_body,
        out_shape=(
            jax.ShapeDtypeStruct((nz, POS_DIM), jnp.float32),
            jax.ShapeDtypeStruct((nz, c), jnp.float32),
        ),
    )(s2, norm_self_gamma.reshape(1, c), norm_self_beta.reshape(1, c),
      zw, z_positions, mlp_w1, mlp_b1.reshape(1, c_mlp),
      mlp_bn_gamma.reshape(1, c_mlp), mlp_bn_beta.reshape(1, c_mlp),
      w2_pos, w2_w, b2_pos, b2_w)
    return out_pos, out_w


# speedup vs baseline: 1.2015x; 1.1375x over previous
import functools

import jax
import jax.numpy as jnp
from jax.experimental import pallas as pl
from jax.experimental.pallas import tpu as pltpu

POS_DIM = 3
EPS = 1e-5
SIGMA = 0.5


def _compw_body(ew_ref, kw_ref, out_ref):
    out_ref[0] = jnp.dot(ew_ref[...], kw_ref[0],
                         preferred_element_type=jnp.float32)


def _make_comp_w(weights, kw):
    n, c = weights.shape
    k = kw.shape[0]
    return pl.pallas_call(
        _compw_body,
        grid=(k,),
        in_specs=[
            pl.BlockSpec((n, c), lambda i: (0, 0)),
            pl.BlockSpec((1, c, c), lambda i: (i, 0, 0)),
        ],
        out_specs=pl.BlockSpec((1, n, c), lambda i: (i, 0, 0)),
        out_shape=jax.ShapeDtypeStruct((k, n, c), jnp.float32),
    )(weights, kw)


def _sample_body(active_ref, jeff_ref, qpos_ref, qb_ref, cposT_ref, cb_ref,
                 cw_ref, kpos_ref, out_ref, *, inv2s2, k):
    i = pl.program_id(0)
    j = pl.program_id(1)

    @pl.when(j == 0)
    def _():
        out_ref[...] = jnp.zeros_like(out_ref)

    @pl.when(active_ref[i, j] != 0)
    def _():
        zp = qpos_ref[...]
        epT = cposT_ref[...]
        kp = kpos_ref[:, 0, :]
        zb = qb_ref[0]
        eb = cb_ref[0]
        mask = zb.T == eb
        z2 = jnp.sum(zp * zp, axis=1)[:, None]
        slabs = []
        for kk in range(k):
            ptT = epT + kp[kk, :][:, None]
            d2 = (z2 + jnp.sum(ptT * ptT, axis=0)[None, :]
                  - 2.0 * jnp.dot(zp, ptT))
            slabs.append(jnp.where(mask, jnp.exp(-inv2s2 * d2), 0.0))
        kern = jnp.concatenate(slabs, axis=1)
        cw = cw_ref[...].reshape(k * cw_ref.shape[1], cw_ref.shape[2])
        out_ref[...] += jnp.dot(kern, cw, preferred_element_type=jnp.float32)


def _block_meta(q_batch, c_batch, bz, be):
    gi = q_batch.shape[0] // bz
    gj = c_batch.shape[0] // be
    qb = q_batch.reshape(gi, bz)
    cb = c_batch.reshape(gj, be)
    qmin, qmax = qb[:, 0], qb[:, -1]
    cmin, cmax = cb[:, 0], cb[:, -1]
    active = ((cmin[None, :] <= qmax[:, None])
              & (qmin[:, None] <= cmax[None, :])).astype(jnp.int32)
    idx = jnp.where(active == 1, jnp.arange(gj, dtype=jnp.int32)[None, :], -1)
    jeff = jnp.maximum(jax.lax.cummax(idx, axis=1), 0).astype(jnp.int32)
    return active, jeff


def _sample(q_pos, q_batch, c_pos, c_batch, comp_w, kpos, sigma,
            bz=512, be=256):
    nq = q_pos.shape[0]
    nc = c_pos.shape[0]
    k, _, c = comp_w.shape
    gi, gj = nq // bz, nc // be
    active, jeff = _block_meta(q_batch, c_batch, bz, be)
    qb = q_batch.reshape(gi, 1, bz)
    cb = c_batch.reshape(gj, 1, be)
    c_posT = c_pos.T
    kpos3 = kpos.reshape(k, 1, POS_DIM)
    grid_spec = pltpu.PrefetchScalarGridSpec(
        num_scalar_prefetch=2,
        grid=(gi, gj),
        in_specs=[
            pl.BlockSpec((bz, POS_DIM), lambda i, j, act, jef: (i, 0)),
            pl.BlockSpec((1, 1, bz), lambda i, j, act, jef: (i, 0, 0)),
            pl.BlockSpec((POS_DIM, be),
                         lambda i, j, act, jef: (0, jef[i, j])),
            pl.BlockSpec((1, 1, be),
                         lambda i, j, act, jef: (jef[i, j], 0, 0)),
            pl.BlockSpec((k, be, c),
                         lambda i, j, act, jef: (0, jef[i, j], 0)),
            pl.BlockSpec((k, 1, POS_DIM), lambda i, j, act, jef: (0, 0, 0)),
        ],
        out_specs=pl.BlockSpec((bz, c), lambda i, j, act, jef: (i, 0)),
    )
    return pl.pallas_call(
        functools.partial(_sample_body, inv2s2=1.0 / (2.0 * sigma * sigma),
                          k=k),
        grid_spec=grid_spec,
        out_shape=jax.ShapeDtypeStruct((nq, c), jnp.float32),
    )(active, jeff, q_pos, qb, c_posT, cb, comp_w, kpos3)


def _bn(x, gamma, beta):
    x = jnp.where(x >= 0, x, 0.01 * x)
    m = jnp.mean(x, axis=0, keepdims=True)
    v = jnp.mean((x - m) ** 2, axis=0, keepdims=True)
    return (x - m) * jax.lax.rsqrt(v + EPS) * gamma + beta


def _bn_compw_body(x_ref, g_ref, b_ref, base_ref, kw_ref, zw_ref, cw_ref,
                   *, k):
    zw = base_ref[...] + _bn(x_ref[...], g_ref[...], b_ref[...])
    zw_ref[...] = zw
    for kk in range(k):
        cw_ref[kk] = jnp.dot(zw, kw_ref[kk],
                             preferred_element_type=jnp.float32)


def _bn_compw(x, gamma, beta, base, kw):
    n, c = x.shape
    k = kw.shape[0]
    return pl.pallas_call(
        functools.partial(_bn_compw_body, k=k),
        out_shape=(
            jax.ShapeDtypeStruct((n, c), jnp.float32),
            jax.ShapeDtypeStruct((k, n, c), jnp.float32),
        ),
    )(x, gamma.reshape(1, c), beta.reshape(1, c), base, kw)


def _mlp_body(x_ref, g2_ref, b2s_ref, zwprev_ref, zpos_ref, w1_ref, b1_ref,
              g_ref, bt_ref, w2p_ref, w2w_ref, b2p_ref, b2w_ref,
              opos_ref, ow_ref):
    zw = zwprev_ref[...] + _bn(x_ref[...], g2_ref[...], b2s_ref[...])
    h = jnp.dot(zw, w1_ref[...], preferred_element_type=jnp.float32)
    h = h + b1_ref[...]
    h = jnp.where(h >= 0, h, 0.01 * h)
    m = jnp.mean(h, axis=0, keepdims=True)
    v = jnp.mean((h - m) ** 2, axis=0, keepdims=True)
    h = (h - m) * jax.lax.rsqrt(v + EPS) * g_ref[...] + bt_ref[...]
    dpos = jnp.dot(h, w2p_ref[...], preferred_element_type=jnp.float32)
    dpos = dpos + b2p_ref[...]
    dw = jnp.dot(h, w2w_ref[...], preferred_element_type=jnp.float32)
    dw = dw + b2w_ref[...]
    opos_ref[...] = zpos_ref[...] + dpos[:, :POS_DIM]
    ow_ref[...] = zw + dw


def kernel(z_positions, z_weights, z_batch, e_positions, e_weights, e_batch,
           cross_kpos, cross_kw, norm_cross_gamma, norm_cross_beta,
           self_kpos, self_kw, norm_self_gamma, norm_self_beta,
           mlp_w1, mlp_b1, mlp_bn_gamma, mlp_bn_beta, mlp_w2, mlp_b2):
    nz, c = z_weights.shape
    c_mlp = mlp_w1.shape[1]

    cw1 = _make_comp_w(e_weights, cross_kw)
    s1 = _sample(z_positions, z_batch, e_positions, e_batch, cw1,
                 cross_kpos, SIGMA)
    zw, cw2 = _bn_compw(s1, norm_cross_gamma, norm_cross_beta, z_weights,
                        self_kw)
    s2 = _sample(z_positions, z_batch, z_positions, z_batch, cw2,
                 self_kpos, SIGMA)

    w2_pos = jnp.zeros((c_mlp, c), jnp.float32).at[:, :POS_DIM].set(
        mlp_w2[:, :POS_DIM])
    b2_pos = jnp.zeros((1, c), jnp.float32).at[0, :POS_DIM].set(
        mlp_b2[:POS_DIM])
    w2_w = mlp_w2[:, POS_DIM:]
    b2_w = mlp_b2[POS_DIM:].reshape(1, c)

    out_pos, out_w = pl.pallas_call(
        _mlp_body,
        out_shape=(
            jax.ShapeDtypeStruct((nz, POS_DIM), jnp.float32),
            jax.ShapeDtypeStruct((nz, c), jnp.float32),
        ),
    )(s2, norm_self_gamma.reshape(1, c), norm_self_beta.reshape(1, c),
      zw, z_positions, mlp_w1, mlp_b1.reshape(1, c_mlp),
      mlp_bn_gamma.reshape(1, c_mlp), mlp_bn_beta.reshape(1, c_mlp),
      w2_pos, w2_w, b2_pos, b2_w)
    return out_pos, out_w


# speedup vs baseline: 1.7485x; 1.4553x over previous
import functools

import jax
import jax.numpy as jnp
from jax.experimental import pallas as pl
from jax.experimental.pallas import tpu as pltpu

POS_DIM = 3
EPS = 1e-5
SIGMA = 0.5
BZ = 512
BE = 256


def _leaky(x):
    return jnp.where(x >= 0, x, 0.01 * x)


def _bn(x, gamma, beta):
    m = jnp.mean(x, axis=0, keepdims=True)
    v = jnp.mean((x - m) ** 2, axis=0, keepdims=True)
    return (x - m) * jax.lax.rsqrt(v + EPS) * gamma + beta


def _body(jlo1_ref, jhi1_ref, jlo2_ref, jhi2_ref,
          zp_ref, zpT_ref, epT_ref, zbc_ref, zbr_ref, ebr_ref,
          ew_ref, kw1_ref, kw2_ref, kp1_ref, kp2_ref,
          zw0_ref, g1_ref, b1_ref, g2_ref, b2_ref,
          w1_ref, bb1_ref, gm_ref, bm_ref,
          w2p_ref, w2w_ref, b2p_ref, b2w_ref,
          opos_ref, ow_ref, cw_s, s_s, zw_s, *, gi, gj, k, c, inv2s2):

    def sample(cposT_ref, cbr_ref, jlo_ref, jhi_ref, kp_ref):
        kp = kp_ref[:, 0, :]
        for i in range(gi):
            zp = zp_ref[i * BZ:(i + 1) * BZ, :]
            zbc = zbc_ref[i * BZ:(i + 1) * BZ, :]
            z2 = jnp.sum(zp * zp, axis=1)[:, None]

            def jbody(j, acc):
                epT = cposT_ref[:, pl.ds(j * BE, BE)]
                eb = cbr_ref[:, pl.ds(j * BE, BE)]
                mask = zbc == eb
                slabs = []
                for kk in range(k):
                    ptT = epT + kp[kk, :][:, None]
                    d2 = (z2 + jnp.sum(ptT * ptT, axis=0)[None, :]
                          - 2.0 * jnp.dot(zp, ptT))
                    slabs.append(jnp.where(mask, jnp.exp(-inv2s2 * d2), 0.0))
                kern = jnp.concatenate(slabs, axis=1)
                cwj = jnp.concatenate(
                    [cw_s[kk, pl.ds(j * BE, BE), :] for kk in range(k)],
                    axis=0)
                return acc + jnp.dot(kern, cwj,
                                     preferred_element_type=jnp.float32)

            acc = jax.lax.fori_loop(
                jlo_ref[i], jhi_ref[i] + 1, jbody,
                jnp.zeros((BZ, c), jnp.float32))
            s_s[i * BZ:(i + 1) * BZ, :] = acc

    ew = ew_ref[...]
    for kk in range(k):
        cw_s[kk] = jnp.dot(ew, kw1_ref[kk], preferred_element_type=jnp.float32)
    sample(epT_ref, ebr_ref, jlo1_ref, jhi1_ref, kp1_ref)

    zw = zw0_ref[...] + _bn(_leaky(s_s[...]), g1_ref[...], b1_ref[...])
    zw_s[...] = zw
    for kk in range(k):
        cw_s[kk] = jnp.dot(zw, kw2_ref[kk], preferred_element_type=jnp.float32)
    sample(zpT_ref, zbr_ref, jlo2_ref, jhi2_ref, kp2_ref)

    zw2 = zw_s[...] + _bn(_leaky(s_s[...]), g2_ref[...], b2_ref[...])
    h = jnp.dot(zw2, w1_ref[...], preferred_element_type=jnp.float32)
    h = _bn(_leaky(h + bb1_ref[...]), gm_ref[...], bm_ref[...])
    dpos = jnp.dot(h, w2p_ref[...], preferred_element_type=jnp.float32)
    dpos = dpos + b2p_ref[...]
    dw = jnp.dot(h, w2w_ref[...], preferred_element_type=jnp.float32)
    dw = dw + b2w_ref[...]
    opos_ref[...] = zp_ref[...] + dpos[:, :POS_DIM]
    ow_ref[...] = zw2 + dw


def _ranges(q_batch, c_batch, bz, be):
    gi = q_batch.shape[0] // bz
    gj = c_batch.shape[0] // be
    qb = q_batch.reshape(gi, bz)
    cb = c_batch.reshape(gj, be)
    qmin, qmax = qb[:, 0], qb[:, -1]
    cmin, cmax = cb[:, 0], cb[:, -1]
    jlo = jnp.sum((cmax[None, :] < qmin[:, None]).astype(jnp.int32), axis=1)
    jhi = (gj - 1
           - jnp.sum((cmin[None, :] > qmax[:, None]).astype(jnp.int32),
                     axis=1))
    return jlo.astype(jnp.int32), jhi.astype(jnp.int32)


def kernel(z_positions, z_weights, z_batch, e_positions, e_weights, e_batch,
           cross_kpos, cross_kw, norm_cross_gamma, norm_cross_beta,
           self_kpos, self_kw, norm_self_gamma, norm_self_beta,
           mlp_w1, mlp_b1, mlp_bn_gamma, mlp_bn_beta, mlp_w2, mlp_b2):
    nz, c = z_weights.shape
    ne = e_weights.shape[0]
    k = cross_kpos.shape[0]
    c_mlp = mlp_w1.shape[1]
    gi, gj = nz // BZ, ne // BE

    jlo1, jhi1 = _ranges(z_batch, e_batch, BZ, BE)
    jlo2, jhi2 = _ranges(z_batch, z_batch, BZ, BE)

    w2_pos = jnp.zeros((c_mlp, c), jnp.float32).at[:, :POS_DIM].set(
        mlp_w2[:, :POS_DIM])
    b2_pos = jnp.zeros((1, c), jnp.float32).at[0, :POS_DIM].set(
        mlp_b2[:POS_DIM])
    w2_w = mlp_w2[:, POS_DIM:]
    b2_w = mlp_b2[POS_DIM:].reshape(1, c)

    full = lambda *shape: pl.BlockSpec(
        shape, (lambda g, a1, a2, a3, a4: tuple(0 for _ in shape)))
    grid_spec = pltpu.PrefetchScalarGridSpec(
        num_scalar_prefetch=4,
        grid=(1,),
        in_specs=[
            full(nz, POS_DIM),
            full(POS_DIM, nz),
            full(POS_DIM, ne),
            full(nz, 1),
            full(1, nz),
            full(1, ne),
            full(ne, c),
            full(k, c, c),
            full(k, c, c),
            full(k, 1, POS_DIM),
            full(k, 1, POS_DIM),
            full(nz, c),
            full(1, c), full(1, c),
            full(1, c), full(1, c),
            full(c, c_mlp),
            full(1, c_mlp),
            full(1, c_mlp), full(1, c_mlp),
            full(c_mlp, c),
            full(c_mlp, c),
            full(1, c), full(1, c),
        ],
        out_specs=[
            pl.BlockSpec((nz, POS_DIM), lambda g, a1, a2, a3, a4: (0, 0)),
            pl.BlockSpec((nz, c), lambda g, a1, a2, a3, a4: (0, 0)),
        ],
        scratch_shapes=[
            pltpu.VMEM((k, ne, c), jnp.float32),
            pltpu.VMEM((nz, c), jnp.float32),
            pltpu.VMEM((nz, c), jnp.float32),
        ],
    )
    out_pos, out_w = pl.pallas_call(
        functools.partial(_body, gi=gi, gj=gj, k=k, c=c,
                          inv2s2=1.0 / (2.0 * SIGMA * SIGMA)),
        grid_spec=grid_spec,
        out_shape=(
            jax.ShapeDtypeStruct((nz, POS_DIM), jnp.float32),
            jax.ShapeDtypeStruct((nz, c), jnp.float32),
        ),
    )(jlo1, jhi1, jlo2, jhi2,
      z_positions, z_positions.T, e_positions.T,
      z_batch.reshape(nz, 1), z_batch.reshape(1, nz), e_batch.reshape(1, ne),
      e_weights, cross_kw, self_kw,
      cross_kpos.reshape(k, 1, POS_DIM), self_kpos.reshape(k, 1, POS_DIM),
      z_weights,
      norm_cross_gamma.reshape(1, c), norm_cross_beta.reshape(1, c),
      norm_self_gamma.reshape(1, c), norm_self_beta.reshape(1, c),
      mlp_w1, mlp_b1.reshape(1, c_mlp),
      mlp_bn_gamma.reshape(1, c_mlp), mlp_bn_beta.reshape(1, c_mlp),
      w2_pos, w2_w, b2_pos, b2_w)
    return out_pos, out_w


# speedup vs baseline: 1.8039x; 1.0317x over previous
import functools

import jax
import jax.numpy as jnp
from jax.experimental import pallas as pl
from jax.experimental.pallas import tpu as pltpu

POS_DIM = 3
EPS = 1e-5
SIGMA = 0.5
BZ = 256
BE = 256


def _leaky(x):
    return jnp.where(x >= 0, x, 0.01 * x)


def _bn(x, gamma, beta):
    m = jnp.mean(x, axis=0, keepdims=True)
    v = jnp.mean((x - m) ** 2, axis=0, keepdims=True)
    return (x - m) * jax.lax.rsqrt(v + EPS) * gamma + beta


def _body(jlo1_ref, jhi1_ref, jlo2_ref, jhi2_ref,
          zp_ref, zpT_ref, epT_ref, zbc_ref, zbr_ref, ebr_ref,
          ew_ref, kw1_ref, kw2_ref, kp1_ref, kp2_ref,
          zw0_ref, g1_ref, b1_ref, g2_ref, b2_ref,
          w1_ref, bb1_ref, gm_ref, bm_ref,
          w2p_ref, w2w_ref, b2p_ref, b2w_ref,
          opos_ref, ow_ref, cw_s, s_s, zw_s, *, gi, gj, k, c, inv2s2):

    def sample(cposT_ref, cbr_ref, jlo_ref, jhi_ref, kp_ref):
        kp = kp_ref[:, 0, :]
        for i in range(gi):
            zp = zp_ref[i * BZ:(i + 1) * BZ, :]
            zbc = zbc_ref[i * BZ:(i + 1) * BZ, :]
            z2 = jnp.sum(zp * zp, axis=1)[:, None]

            def jbody(j, acc):
                epT = cposT_ref[:, pl.ds(j * BE, BE)]
                eb = cbr_ref[:, pl.ds(j * BE, BE)]
                mask = zbc == eb
                slabs = []
                for kk in range(k):
                    ptT = epT + kp[kk, :][:, None]
                    d2 = (z2 + jnp.sum(ptT * ptT, axis=0)[None, :]
                          - 2.0 * jnp.dot(zp, ptT))
                    slabs.append(jnp.where(mask, jnp.exp(-inv2s2 * d2), 0.0))
                kern = jnp.concatenate(slabs, axis=1)
                cwj = jnp.concatenate(
                    [cw_s[kk, pl.ds(j * BE, BE), :] for kk in range(k)],
                    axis=0)
                return acc + jnp.dot(kern, cwj,
                                     preferred_element_type=jnp.float32)

            acc = jax.lax.fori_loop(
                jlo_ref[i], jhi_ref[i] + 1, jbody,
                jnp.zeros((BZ, c), jnp.float32))
            s_s[i * BZ:(i + 1) * BZ, :] = acc

    ew = ew_ref[...]
    for kk in range(k):
        cw_s[kk] = jnp.dot(ew, kw1_ref[kk], preferred_element_type=jnp.float32)
    sample(epT_ref, ebr_ref, jlo1_ref, jhi1_ref, kp1_ref)

    zw = zw0_ref[...] + _bn(_leaky(s_s[...]), g1_ref[...], b1_ref[...])
    zw_s[...] = zw
    for kk in range(k):
        cw_s[kk] = jnp.dot(zw, kw2_ref[kk], preferred_element_type=jnp.float32)
    sample(zpT_ref, zbr_ref, jlo2_ref, jhi2_ref, kp2_ref)

    zw2 = zw_s[...] + _bn(_leaky(s_s[...]), g2_ref[...], b2_ref[...])
    h = jnp.dot(zw2, w1_ref[...], preferred_element_type=jnp.float32)
    h = _bn(_leaky(h + bb1_ref[...]), gm_ref[...], bm_ref[...])
    dpos = jnp.dot(h, w2p_ref[...], preferred_element_type=jnp.float32)
    dpos = dpos + b2p_ref[...]
    dw = jnp.dot(h, w2w_ref[...], preferred_element_type=jnp.float32)
    dw = dw + b2w_ref[...]
    opos_ref[...] = zp_ref[...] + dpos[:, :POS_DIM]
    ow_ref[...] = zw2 + dw


def _ranges(q_batch, c_batch, bz, be):
    gi = q_batch.shape[0] // bz
    gj = c_batch.shape[0] // be
    qb = q_batch.reshape(gi, bz)
    cb = c_batch.reshape(gj, be)
    qmin, qmax = qb[:, 0], qb[:, -1]
    cmin, cmax = cb[:, 0], cb[:, -1]
    jlo = jnp.sum((cmax[None, :] < qmin[:, None]).astype(jnp.int32), axis=1)
    jhi = (gj - 1
           - jnp.sum((cmin[None, :] > qmax[:, None]).astype(jnp.int32),
                     axis=1))
    return jlo.astype(jnp.int32), jhi.astype(jnp.int32)


def kernel(z_positions, z_weights, z_batch, e_positions, e_weights, e_batch,
           cross_kpos, cross_kw, norm_cross_gamma, norm_cross_beta,
           self_kpos, self_kw, norm_self_gamma, norm_self_beta,
           mlp_w1, mlp_b1, mlp_bn_gamma, mlp_bn_beta, mlp_w2, mlp_b2):
    nz, c = z_weights.shape
    ne = e_weights.shape[0]
    k = cross_kpos.shape[0]
    c_mlp = mlp_w1.shape[1]
    gi, gj = nz // BZ, ne // BE

    jlo1, jhi1 = _ranges(z_batch, e_batch, BZ, BE)
    jlo2, jhi2 = _ranges(z_batch, z_batch, BZ, BE)

    w2_pos = jnp.zeros((c_mlp, c), jnp.float32).at[:, :POS_DIM].set(
        mlp_w2[:, :POS_DIM])
    b2_pos = jnp.zeros((1, c), jnp.float32).at[0, :POS_DIM].set(
        mlp_b2[:POS_DIM])
    w2_w = mlp_w2[:, POS_DIM:]
    b2_w = mlp_b2[POS_DIM:].reshape(1, c)

    full = lambda *shape: pl.BlockSpec(
        shape, (lambda g, a1, a2, a3, a4: tuple(0 for _ in shape)))
    grid_spec = pltpu.PrefetchScalarGridSpec(
        num_scalar_prefetch=4,
        grid=(1,),
        in_specs=[
            full(nz, POS_DIM),
            full(POS_DIM, nz),
            full(POS_DIM, ne),
            full(nz, 1),
            full(1, nz),
            full(1, ne),
            full(ne, c),
            full(k, c, c),
            full(k, c, c),
            full(k, 1, POS_DIM),
            full(k, 1, POS_DIM),
            full(nz, c),
            full(1, c), full(1, c),
            full(1, c), full(1, c),
            full(c, c_mlp),
            full(1, c_mlp),
            full(1, c_mlp), full(1, c_mlp),
            full(c_mlp, c),
            full(c_mlp, c),
            full(1, c), full(1, c),
        ],
        out_specs=[
            pl.BlockSpec((nz, POS_DIM), lambda g, a1, a2, a3, a4: (0, 0)),
            pl.BlockSpec((nz, c), lambda g, a1, a2, a3, a4: (0, 0)),
        ],
        scratch_shapes=[
            pltpu.VMEM((k, ne, c), jnp.float32),
            pltpu.VMEM((nz, c), jnp.float32),
            pltpu.VMEM((nz, c), jnp.float32),
        ],
    )
    out_pos, out_w = pl.pallas_call(
        functools.partial(_body, gi=gi, gj=gj, k=k, c=c,
                          inv2s2=1.0 / (2.0 * SIGMA * SIGMA)),
        grid_spec=grid_spec,
        out_shape=(
            jax.ShapeDtypeStruct((nz, POS_DIM), jnp.float32),
            jax.ShapeDtypeStruct((nz, c), jnp.float32),
        ),
    )(jlo1, jhi1, jlo2, jhi2,
      z_positions, z_positions.T, e_positions.T,
      z_batch.reshape(nz, 1), z_batch.reshape(1, nz), e_batch.reshape(1, ne),
      e_weights, cross_kw, self_kw,
      cross_kpos.reshape(k, 1, POS_DIM), self_kpos.reshape(k, 1, POS_DIM),
      z_weights,
      norm_cross_gamma.reshape(1, c), norm_cross_beta.reshape(1, c),
      norm_self_gamma.reshape(1, c), norm_self_beta.reshape(1, c),
      mlp_w1, mlp_b1.reshape(1, c_mlp),
      mlp_bn_gamma.reshape(1, c_mlp), mlp_bn_beta.reshape(1, c_mlp),
      w2_pos, w2_w, b2_pos, b2_w)
    return out_pos, out_w
